# baseline (speedup 1.0000x reference)
import jax
import jax.numpy as jnp
from jax.experimental import pallas as pl
from jax.experimental.pallas import tpu as pltpu

_TOP_P = 0.9
_ROW_BLOCK = 8
_CHUNK = 12800


def _sortable_key(x):
    bi = jax.lax.bitcast_convert_type(x, jnp.int32)
    return bi ^ (jnp.right_shift(bi, 31) & jnp.int32(0x7FFFFFFF))


def _mid(lo, hi):
    return (lo >> 1) + (hi >> 1) + (lo & hi & 1)


def _csum(fn, V):
    parts = [jnp.sum(fn(s, min(s + _CHUNK, V)), axis=-1, keepdims=True)
             for s in range(0, V, _CHUNK)]
    while len(parts) > 1:
        parts = [a + b for a, b in zip(parts[::2], parts[1::2])] + (
            [parts[-1]] if len(parts) % 2 else [])
    return parts[0]


def _sample_kernel(x_ref, g_ref, k_ref, probs_ref, tok_ref):
    x = x_ref[...]
    k = k_ref[0, 0]
    B, V = x.shape

    K = _sortable_key(x)
    m = jnp.max(x, axis=-1, keepdims=True)
    E = jnp.exp(x - m)

    imin = jnp.int32(jnp.iinfo(jnp.int32).min)
    imax = jnp.int32(jnp.iinfo(jnp.int32).max)

    def not_converged(carry):
        lo, hi = carry
        return jnp.any(hi > lo + 1)

    def _sum3(fn, mids):
        accs = [None, None, None]
        for s in range(0, V, _CHUNK):
            e = min(s + _CHUNK, V)
            for j in range(3):
                p = jnp.sum(fn(s, e, mids[j]), axis=-1, keepdims=True)
                accs[j] = p if accs[j] is None else accs[j] + p
        return accs

    def kth_body(carry):
        lo, hi = carry
        mid_m = _mid(lo, hi)
        mid_l = _mid(lo, mid_m)
        mid_h = _mid(mid_m, hi)
        c_l, c_m, c_h = _sum3(
            lambda s, e, t: (K[:, s:e] >= t).astype(jnp.int32),
            (mid_l, mid_m, mid_h))
        p_l, p_m, p_h = c_l >= k, c_m >= k, c_h >= k
        lo2 = jnp.where(p_h, mid_h,
                        jnp.where(p_m, mid_m, jnp.where(p_l, mid_l, lo)))
        hi2 = jnp.where(~p_l, mid_l,
                        jnp.where(~p_m, mid_m, jnp.where(~p_h, mid_h, hi)))
        return lo2, hi2

    kth_key, _ = jax.lax.while_loop(
        not_converged, kth_body,
        (jnp.full((B, 1), imin, jnp.int32), jnp.full((B, 1), imax, jnp.int32)),
    )

    T = _csum(lambda s, e: jnp.where(K[:, s:e] >= kth_key, E[:, s:e],
                                     jnp.float32(0.0)), V)
    cap = jnp.float32(_TOP_P) * T
    key_max = _sortable_key(m)

    def nuc_body(carry):
        lo, hi = carry
        mid_m = _mid(lo, hi)
        mid_l = _mid(lo, mid_m)
        mid_h = _mid(mid_m, hi)
        f_l, f_m, f_h = _sum3(
            lambda s, e, t: jnp.where(K[:, s:e] > t, E[:, s:e],
                                      jnp.float32(0.0)),
            (mid_l, mid_m, mid_h))
        r_l, r_m, r_h = f_l <= cap, f_m <= cap, f_h <= cap
        hi2 = jnp.where(r_l, mid_l,
                        jnp.where(r_m, mid_m, jnp.where(r_h, mid_h, hi)))
        lo2 = jnp.where(~r_h, mid_h,
                        jnp.where(~r_m, mid_m, jnp.where(~r_l, mid_l, lo)))
        return lo2, hi2

    _, vstar_key = jax.lax.while_loop(
        not_converged, nuc_body, (kth_key - 1, key_max + 1),
    )

    mask = K >= jnp.maximum(kth_key, vstar_key)
    EF = jnp.where(mask, E, jnp.float32(0.0))
    Z = _csum(lambda s, e: EF[:, s:e], V)
    probs_ref[...] = EF / Z

    score = jnp.where(mask, x + g_ref[...], jnp.float32(-jnp.inf))
    tok_ref[...] = jnp.argmax(score, axis=-1).astype(jnp.int32).reshape(B, 1)


def kernel(logits, top_k):
    B, V = logits.shape
    g = jax.random.gumbel(jax.random.key(42), (B, V), logits.dtype)
    k_arr = jnp.full((8, 128), top_k, jnp.int32)
    probs, tok = pl.pallas_call(
        _sample_kernel,
        grid=(B // _ROW_BLOCK,),
        in_specs=[
            pl.BlockSpec((_ROW_BLOCK, V), lambda i: (i, 0)),
            pl.BlockSpec((_ROW_BLOCK, V), lambda i: (i, 0)),
            pl.BlockSpec((8, 128), lambda i: (0, 0)),
        ],
        out_specs=[
            pl.BlockSpec((_ROW_BLOCK, V), lambda i: (i, 0)),
            pl.BlockSpec((_ROW_BLOCK, 1), lambda i: (i, 0)),
        ],
        out_shape=[
            jax.ShapeDtypeStruct((B, V), logits.dtype),
            jax.ShapeDtypeStruct((B, 1), jnp.int32),
        ],
        compiler_params=pltpu.CompilerParams(
            dimension_semantics=("arbitrary",),
        ),
    )(logits, g, k_arr)
    return probs, tok.reshape(B)


# speedup vs baseline: 1.2074x; 1.2074x over previous
import jax
import jax.numpy as jnp
from jax.experimental import pallas as pl
from jax.experimental.pallas import tpu as pltpu

_TOP_P = 0.9
_ROW_BLOCK = 8
_CHUNK = 12800


def _sortable_key(x):
    bi = jax.lax.bitcast_convert_type(x, jnp.int32)
    return bi ^ (jnp.right_shift(bi, 31) & jnp.int32(0x7FFFFFFF))


def _mid(lo, hi):
    return (lo >> 1) + (hi >> 1) + (lo & hi & 1)


def _csum(fn, V):
    parts = [jnp.sum(fn(s, min(s + _CHUNK, V)), axis=-1, keepdims=True)
             for s in range(0, V, _CHUNK)]
    while len(parts) > 1:
        parts = [a + b for a, b in zip(parts[::2], parts[1::2])] + (
            [parts[-1]] if len(parts) % 2 else [])
    return parts[0]


def _sample_kernel(x_ref, g_ref, k_ref, probs_ref, tok_ref):
    x = x_ref[...]
    k = k_ref[0, 0]
    B, V = x.shape

    K = _sortable_key(x)
    m = jnp.max(x, axis=-1, keepdims=True)
    E = jnp.exp(x - m)

    imin = jnp.int32(jnp.iinfo(jnp.int32).min)
    imax = jnp.int32(jnp.iinfo(jnp.int32).max)

    def kth_cond(carry):
        lo, hi, c_lo = carry
        return jnp.any((hi > lo + 1) & (c_lo != k))

    def kth_body(carry):
        lo, hi, c_lo = carry
        mid = _mid(lo, hi)
        cnt = _csum(lambda s, e: (K[:, s:e] >= mid).astype(jnp.int32), V)
        pred = cnt >= k
        return (jnp.where(pred, mid, lo), jnp.where(pred, hi, mid),
                jnp.where(pred, cnt, c_lo))

    kth_key, _, _ = jax.lax.while_loop(
        kth_cond, kth_body,
        (jnp.full((B, 1), imin, jnp.int32), jnp.full((B, 1), imax, jnp.int32),
         jnp.full((B, 1), jnp.int32(V), jnp.int32)),
    )

    T = _csum(lambda s, e: jnp.where(K[:, s:e] >= kth_key, E[:, s:e],
                                     jnp.float32(0.0)), V)
    cap = jnp.float32(_TOP_P) * T
    key_max = _sortable_key(m)

    def nuc_cond(carry):
        lo, hi, f_lo, f_hi = carry
        return jnp.any((hi > lo + 1) & (f_lo != f_hi))

    def nuc_body(carry):
        lo, hi, f_lo, f_hi = carry
        mid = _mid(lo, hi)
        fex = _csum(lambda s, e: jnp.where(K[:, s:e] > mid, E[:, s:e],
                                           jnp.float32(0.0)), V)
        pred = fex <= cap
        return (jnp.where(pred, lo, mid), jnp.where(pred, mid, hi),
                jnp.where(pred, f_lo, fex), jnp.where(pred, fex, f_hi))

    _, vstar_key, _, _ = jax.lax.while_loop(
        nuc_cond, nuc_body,
        (kth_key - 1, key_max + 1, T, jnp.zeros_like(T)),
    )

    mask = K >= jnp.maximum(kth_key, vstar_key)
    EF = jnp.where(mask, E, jnp.float32(0.0))
    Z = _csum(lambda s, e: EF[:, s:e], V)
    probs_ref[...] = EF / Z

    score = jnp.where(mask, x + g_ref[...], jnp.float32(-jnp.inf))
    tok_ref[...] = jnp.argmax(score, axis=-1).astype(jnp.int32).reshape(B, 1)


def kernel(logits, top_k):
    B, V = logits.shape
    g = jax.random.gumbel(jax.random.key(42), (B, V), logits.dtype)
    k_arr = jnp.full((8, 128), top_k, jnp.int32)
    probs, tok = pl.pallas_call(
        _sample_kernel,
        grid=(B // _ROW_BLOCK,),
        in_specs=[
            pl.BlockSpec((_ROW_BLOCK, V), lambda i: (i, 0)),
            pl.BlockSpec((_ROW_BLOCK, V), lambda i: (i, 0)),
            pl.BlockSpec((8, 128), lambda i: (0, 0)),
        ],
        out_specs=[
            pl.BlockSpec((_ROW_BLOCK, V), lambda i: (i, 0)),
            pl.BlockSpec((_ROW_BLOCK, 1), lambda i: (i, 0)),
        ],
        out_shape=[
            jax.ShapeDtypeStruct((B, V), logits.dtype),
            jax.ShapeDtypeStruct((B, 1), jnp.int32),
        ],
        compiler_params=pltpu.CompilerParams(
            dimension_semantics=("arbitrary",),
        ),
    )(logits, g, k_arr)
    return probs, tok.reshape(B)
